# baseline (speedup 1.0000x reference)
import jax
import jax.numpy as jnp
from jax.experimental import pallas as pl
from jax.experimental.pallas import tpu as pltpu

BATCH = 16384
HIDDEN = 256
BLOCK_ROWS = 8192


def _body(bias_ref, a_ref, w_ref, o_ref):
    h = jnp.clip(a_ref[...], 0.0, 1.0).astype(jnp.bfloat16)
    res = jax.lax.dot_general(
        w_ref[...].astype(jnp.bfloat16), h, (((1,), (1,)), ((), ())),
        preferred_element_type=jnp.float32)
    o_ref[...] = res[0] + bias_ref[0]


def kernel(accumulator, output_weights, output_bias):
    bias = jnp.reshape(output_bias, (1,)).astype(jnp.float32)
    w2d = jnp.reshape(output_weights, (1, HIDDEN))
    grid = (BATCH // BLOCK_ROWS,)
    out = pl.pallas_call(
        _body,
        grid=grid,
        in_specs=[
            pl.BlockSpec(memory_space=pltpu.MemorySpace.SMEM),
            pl.BlockSpec((BLOCK_ROWS, HIDDEN), lambda i: (i, 0)),
            pl.BlockSpec((1, HIDDEN), lambda i: (0, 0)),
        ],
        out_specs=pl.BlockSpec((BLOCK_ROWS,), lambda i: (i,)),
        out_shape=jax.ShapeDtypeStruct((BATCH,), jnp.float32),
    )(bias, accumulator, w2d)
    return out


# speedup vs baseline: 1.0024x; 1.0024x over previous
import jax
import jax.numpy as jnp
from jax.experimental import pallas as pl
from jax.experimental.pallas import tpu as pltpu

BATCH = 16384
HIDDEN = 256
BLOCK_ROWS = 8192


def _body(bias_ref, a_ref, w_ref, o_ref):
    h = jnp.clip(a_ref[...].astype(jnp.bfloat16), 0.0, 1.0)
    res = jax.lax.dot_general(
        w_ref[...].astype(jnp.bfloat16), h, (((1,), (1,)), ((), ())),
        preferred_element_type=jnp.float32)
    o_ref[...] = res[0] + bias_ref[0]


def kernel(accumulator, output_weights, output_bias):
    bias = jnp.reshape(output_bias, (1,)).astype(jnp.float32)
    w2d = jnp.reshape(output_weights, (1, HIDDEN))
    grid = (BATCH // BLOCK_ROWS,)
    out = pl.pallas_call(
        _body,
        grid=grid,
        in_specs=[
            pl.BlockSpec(memory_space=pltpu.MemorySpace.SMEM),
            pl.BlockSpec((BLOCK_ROWS, HIDDEN), lambda i: (i, 0)),
            pl.BlockSpec((1, HIDDEN), lambda i: (0, 0)),
        ],
        out_specs=pl.BlockSpec((BLOCK_ROWS,), lambda i: (i,)),
        out_shape=jax.ShapeDtypeStruct((BATCH,), jnp.float32),
    )(bias, accumulator, w2d)
    return out


# speedup vs baseline: 1.0050x; 1.0026x over previous
import jax
import jax.numpy as jnp
from jax.experimental import pallas as pl
from jax.experimental.pallas import tpu as pltpu

BATCH = 16384
HIDDEN = 256
BLOCK_ROWS = 8192


def _body(bias_ref, a_ref, w_ref, o_ref):
    h = jnp.clip(a_ref[...], 0.0, 1.0)
    res = jax.lax.dot_general(
        w_ref[...], h, (((1,), (1,)), ((), ())),
        preferred_element_type=jnp.float32)
    o_ref[...] = res[0] + bias_ref[0]


def kernel(accumulator, output_weights, output_bias):
    bias = jnp.reshape(output_bias, (1,)).astype(jnp.float32)
    w2d = jnp.reshape(output_weights, (1, HIDDEN))
    grid = (BATCH // BLOCK_ROWS,)
    out = pl.pallas_call(
        _body,
        grid=grid,
        in_specs=[
            pl.BlockSpec(memory_space=pltpu.MemorySpace.SMEM),
            pl.BlockSpec((BLOCK_ROWS, HIDDEN), lambda i: (i, 0)),
            pl.BlockSpec((1, HIDDEN), lambda i: (0, 0)),
        ],
        out_specs=pl.BlockSpec((BLOCK_ROWS,), lambda i: (i,)),
        out_shape=jax.ShapeDtypeStruct((BATCH,), jnp.float32),
    )(bias, accumulator, w2d)
    return out
